# baseline (speedup 1.0000x reference)
import functools
import math

import jax
import jax.numpy as jnp
from jax import lax
from jax.experimental import pallas as pl
from jax.experimental.pallas import tpu as pltpu
from jax.experimental.pallas import tpu_sc as plsc

D = 64
IDX_CHUNK = 128
NBUF = 2


def _sc_gather_quads(upack, ipack, uhi2, ihi2, n_workers, chunks):
    B = n_workers * chunks * IDX_CHUNK

    mesh = plsc.VectorSubcoreMesh(core_axis_name="c", subcore_axis_name="s")
    NC = plsc.get_sparse_core_info().num_cores

    @functools.partial(
        pl.kernel,
        out_type=(
            jax.ShapeDtypeStruct((B, 2 * D), jnp.int32),
            jax.ShapeDtypeStruct((B, 2 * D), jnp.int32),
        ),
        mesh=mesh,
        scratch_types=[
            pltpu.VMEM((NBUF, IDX_CHUNK), jnp.int32),
            pltpu.VMEM((NBUF, IDX_CHUNK, 2 * D), jnp.int32),
            pltpu.SemaphoreType.DMA,
            pltpu.SemaphoreType.DMA,
        ],
    )
    def k(upk, ipk, uhi, ihi, uqo, iqo, idx_v, rows_v, sem0, sem1):
        sems = (sem0, sem1)
        wid = lax.axis_index("s") * NC + lax.axis_index("c")
        units = []
        for t in range(2):
            tab = (upk, ipk)[t]
            ids = (uhi, ihi)[t]
            out = (uqo, iqo)[t]
            for c in range(chunks):
                units.append((tab, ids, out, c))

        def issue(j, slot):
            tab, ids, out, c = units[j]
            row = wid * chunks + c
            pltpu.sync_copy(ids.at[row], idx_v.at[slot])
            return pltpu.async_copy(tab.at[idx_v.at[slot]], rows_v.at[slot],
                                    sems[slot])

        def retire(j, slot, cp):
            _, _, out, c = units[j]
            row = wid * chunks + c
            cp.wait()
            pltpu.sync_copy(rows_v.at[slot], out.at[pl.ds(row * IDX_CHUNK,
                                                          IDX_CHUNK)])

        inflight = []
        for j in range(len(units)):
            slot = j % NBUF
            if len(inflight) == NBUF:
                retire(j - NBUF, slot, inflight.pop(0))
            inflight.append(issue(j, slot))
        nu = len(units)
        for i, cp in enumerate(inflight):
            j = nu - len(inflight) + i
            retire(j, j % NBUF, cp)

    return k(upack, ipack, uhi2, ihi2)


def _unpack_sel(q_ref, qsel):
    x01 = jnp.where(qsel < 1, q_ref[:, 0:32], q_ref[:, 32:64])
    x23 = jnp.where(qsel < 3, q_ref[:, 64:96], q_ref[:, 96:128])
    x = jnp.where(qsel < 2, x01, x23)
    even = lax.bitcast_convert_type(x << 16, jnp.float32)
    odd = lax.bitcast_convert_type(
        x & jnp.int32(-65536), jnp.float32)
    return even, odd


def _mlp_body(uq_ref, iq_ref, us_ref, is_ref,
              w0ue_ref, w0uo_ref, w0ie_ref, w0io_ref,
              b0_ref, g0_ref, bt0_ref,
              w1_ref, b1_ref, g1_ref, bt1_ref,
              w2_ref, b2_ref, g2_ref, bt2_ref,
              wo_ref, bo_ref, out_ref):
    inv = 1.0 / math.sqrt(1.0 + 1e-5)
    ue_e, ue_o = _unpack_sel(uq_ref, us_ref[...])
    ie_e, ie_o = _unpack_sel(iq_ref, is_ref[...])
    x = (jnp.dot(ue_e, w0ue_ref[...], preferred_element_type=jnp.float32)
         + jnp.dot(ue_o, w0uo_ref[...], preferred_element_type=jnp.float32)
         + jnp.dot(ie_e, w0ie_ref[...], preferred_element_type=jnp.float32)
         + jnp.dot(ie_o, w0io_ref[...], preferred_element_type=jnp.float32)
         + b0_ref[...])
    x = g0_ref[...] * (jnp.maximum(x, 0.0) * inv) + bt0_ref[...]
    x = jnp.dot(x, w1_ref[...], preferred_element_type=jnp.float32) + b1_ref[...]
    x = g1_ref[...] * (jnp.maximum(x, 0.0) * inv) + bt1_ref[...]
    x = jnp.dot(x, w2_ref[...], preferred_element_type=jnp.float32) + b2_ref[...]
    x = g2_ref[...] * (jnp.maximum(x, 0.0) * inv) + bt2_ref[...]
    o = jnp.dot(x, wo_ref[...], preferred_element_type=jnp.float32) + bo_ref[...]
    out_ref[...] = jax.nn.sigmoid(o)


def _mlp(uq, iq, usel, isel, W0, b0, g0, bt0, W1, b1, g1, bt1,
         W2, b2, g2, bt2, Wo, bo, block_m):
    B = uq.shape[0]
    grid = (B // block_m,)

    def batch_spec(cols):
        return pl.BlockSpec((block_m, cols), lambda i: (i, 0))

    def full_spec(arr):
        return pl.BlockSpec(arr.shape, lambda i: (0,) * arr.ndim)

    row = lambda v: v.reshape(1, -1)
    args = (uq, iq, usel, isel,
            W0[0:D:2, :], W0[1:D:2, :], W0[D::2, :], W0[D + 1::2, :],
            row(b0), row(g0), row(bt0),
            W1, row(b1), row(g1), row(bt1),
            W2, row(b2), row(g2), row(bt2),
            Wo, row(bo))
    in_specs = ([batch_spec(2 * D), batch_spec(2 * D),
                 batch_spec(1), batch_spec(1)]
                + [full_spec(a) for a in args[4:]])
    return pl.pallas_call(
        _mlp_body,
        grid=grid,
        in_specs=in_specs,
        out_specs=pl.BlockSpec((block_m, 1), lambda i: (i, 0)),
        out_shape=jax.ShapeDtypeStruct((B, 1), jnp.float32),
    )(*args)


def _pack_table(table):
    n = table.shape[0]
    tb = table.astype(jnp.bfloat16).reshape(n, D // 2, 2)
    ti = lax.bitcast_convert_type(tb, jnp.int32)
    return ti.reshape(n // 4, 4 * (D // 2))


def kernel(user_ids, item_ids, user_table, item_table,
           W0, b0, gamma0, beta0,
           W1, b1, gamma1, beta1,
           W2, b2, gamma2, beta2,
           Wo, bo):
    B = user_ids.shape[0]
    info = plsc.get_sparse_core_info()
    n_workers = info.num_cores * info.num_subcores
    chunks = B // (n_workers * IDX_CHUNK)
    upack = _pack_table(user_table)
    ipack = _pack_table(item_table)
    uid = user_ids.astype(jnp.int32)
    iid = item_ids.astype(jnp.int32)
    uhi2 = (uid >> 2).reshape(n_workers * chunks, IDX_CHUNK)
    ihi2 = (iid >> 2).reshape(n_workers * chunks, IDX_CHUNK)
    uq, iq = _sc_gather_quads(upack, ipack, uhi2, ihi2, n_workers, chunks)
    usel = (uid & 3).reshape(B, 1)
    isel = (iid & 3).reshape(B, 1)
    out = _mlp(uq, iq, usel, isel,
               W0, b0, gamma0, beta0, W1, b1, gamma1, beta1,
               W2, b2, gamma2, beta2, Wo, bo, block_m=2048)
    return out.reshape(B)


# speedup vs baseline: 2.9580x; 2.9580x over previous
import functools
import math

import jax
import jax.numpy as jnp
from jax import lax
from jax.experimental import pallas as pl
from jax.experimental.pallas import tpu as pltpu
from jax.experimental.pallas import tpu_sc as plsc

D = 64
IDX_CHUNK = 128
NBUF = 2


def _sc_gather_quads(upack, ipack, uhi2, ihi2, n_workers, chunks):
    B = n_workers * chunks * IDX_CHUNK

    mesh = plsc.VectorSubcoreMesh(core_axis_name="c", subcore_axis_name="s")
    NC = plsc.get_sparse_core_info().num_cores

    @functools.partial(
        pl.kernel,
        out_type=(
            jax.ShapeDtypeStruct((B, 2 * D), jnp.int32),
            jax.ShapeDtypeStruct((B, 2 * D), jnp.int32),
        ),
        mesh=mesh,
        scratch_types=[
            pltpu.VMEM((NBUF, IDX_CHUNK), jnp.int32),
            pltpu.VMEM((NBUF, IDX_CHUNK, 2 * D), jnp.int32),
            pltpu.SemaphoreType.DMA,
            pltpu.SemaphoreType.DMA,
        ],
    )
    def k(upk, ipk, uhi, ihi, uqo, iqo, idx_v, rows_v, sem0, sem1):
        sems = (sem0, sem1)
        wid = lax.axis_index("s") * NC + lax.axis_index("c")
        units = []
        for t in range(2):
            tab = (upk, ipk)[t]
            ids = (uhi, ihi)[t]
            out = (uqo, iqo)[t]
            for c in range(chunks):
                units.append((tab, ids, out, c))

        def issue(j, slot):
            tab, ids, out, c = units[j]
            row = wid * chunks + c
            pltpu.sync_copy(ids.at[row], idx_v.at[slot])
            return pltpu.async_copy(tab.at[idx_v.at[slot]], rows_v.at[slot],
                                    sems[slot])

        def retire(j, slot, cp):
            _, _, out, c = units[j]
            row = wid * chunks + c
            cp.wait()
            pltpu.sync_copy(rows_v.at[slot], out.at[pl.ds(row * IDX_CHUNK,
                                                          IDX_CHUNK)])

        inflight = []
        for j in range(len(units)):
            slot = j % NBUF
            if len(inflight) == NBUF:
                retire(j - NBUF, slot, inflight.pop(0))
            inflight.append(issue(j, slot))
        nu = len(units)
        for i, cp in enumerate(inflight):
            j = nu - len(inflight) + i
            retire(j, j % NBUF, cp)

    return k(upack, ipack, uhi2, ihi2)


PACK_BLK = 2048


def _pack_body(tt_ref, eye_ref, out_ref, *, nfull, tail_q):
    dn = (((0,), (0,)), ((), ()))
    blk_t = lax.dot_general(tt_ref[...], eye_ref[...], dn,
                            preferred_element_type=jnp.float32)
    bits = lax.bitcast_convert_type(blk_t, jnp.int32)
    rne = (bits + jnp.int32(0x7FFF) + ((bits >> 16) & 1)) >> 16
    packed = (rne[:, 32:64] << 16) | (rne[:, 0:32] & jnp.int32(0xFFFF))

    def store(q):
        for u in range(4):
            out_ref[0:q, 32 * u:32 * (u + 1)] = packed[u * q:(u + 1) * q, :]

    if tail_q:
        @pl.when(pl.program_id(0) < nfull)
        def _():
            store(PACK_BLK // 4)

        @pl.when(pl.program_id(0) == nfull)
        def _():
            store(tail_q)
    else:
        store(PACK_BLK // 4)


def _pack_table(tt):
    n = tt.shape[1]
    nfull = n // PACK_BLK
    tail_q = (n - nfull * PACK_BLK) // 4
    eye = jnp.eye(D, dtype=jnp.float32)
    return pl.pallas_call(
        functools.partial(_pack_body, nfull=nfull, tail_q=tail_q),
        grid=(nfull + (1 if tail_q else 0),),
        in_specs=[pl.BlockSpec((D, PACK_BLK), lambda i: (0, i)),
                  pl.BlockSpec((D, D), lambda i: (0, 0))],
        out_specs=pl.BlockSpec((PACK_BLK // 4, 128), lambda i: (i, 0)),
        out_shape=jax.ShapeDtypeStruct((n // 4, 128), jnp.int32),
    )(tt, eye)


def _unpack_sel(q_ref, qsel):
    x01 = jnp.where(qsel < 1, q_ref[:, 0:32], q_ref[:, 32:64])
    x23 = jnp.where(qsel < 3, q_ref[:, 64:96], q_ref[:, 96:128])
    x = jnp.where(qsel < 2, x01, x23)
    lo = lax.bitcast_convert_type(x << 16, jnp.float32)
    hi = lax.bitcast_convert_type(
        x & jnp.int32(-65536), jnp.float32)
    return lo, hi


def _mlp_body(uq_ref, iq_ref, us_ref, is_ref,
              w0ue_ref, w0uo_ref, w0ie_ref, w0io_ref,
              b0_ref, g0_ref, bt0_ref,
              w1_ref, b1_ref, g1_ref, bt1_ref,
              w2_ref, b2_ref, g2_ref, bt2_ref,
              wo_ref, bo_ref, out_ref):
    inv = 1.0 / math.sqrt(1.0 + 1e-5)
    ue_e, ue_o = _unpack_sel(uq_ref, us_ref[...])
    ie_e, ie_o = _unpack_sel(iq_ref, is_ref[...])
    x = (jnp.dot(ue_e, w0ue_ref[...], preferred_element_type=jnp.float32)
         + jnp.dot(ue_o, w0uo_ref[...], preferred_element_type=jnp.float32)
         + jnp.dot(ie_e, w0ie_ref[...], preferred_element_type=jnp.float32)
         + jnp.dot(ie_o, w0io_ref[...], preferred_element_type=jnp.float32)
         + b0_ref[...])
    x = g0_ref[...] * (jnp.maximum(x, 0.0) * inv) + bt0_ref[...]
    x = jnp.dot(x, w1_ref[...], preferred_element_type=jnp.float32) + b1_ref[...]
    x = g1_ref[...] * (jnp.maximum(x, 0.0) * inv) + bt1_ref[...]
    x = jnp.dot(x, w2_ref[...], preferred_element_type=jnp.float32) + b2_ref[...]
    x = g2_ref[...] * (jnp.maximum(x, 0.0) * inv) + bt2_ref[...]
    o = jnp.dot(x, wo_ref[...], preferred_element_type=jnp.float32) + bo_ref[...]
    out_ref[...] = jax.nn.sigmoid(o)


def _mlp(uq, iq, usel, isel, W0, b0, g0, bt0, W1, b1, g1, bt1,
         W2, b2, g2, bt2, Wo, bo, block_m):
    B = uq.shape[0]
    grid = (B // block_m,)

    def batch_spec(cols):
        return pl.BlockSpec((block_m, cols), lambda i: (i, 0))

    def full_spec(arr):
        return pl.BlockSpec(arr.shape, lambda i: (0,) * arr.ndim)

    row = lambda v: v.reshape(1, -1)
    args = (uq, iq, usel, isel,
            W0[0:32, :], W0[32:64, :], W0[64:96, :], W0[96:128, :],
            row(b0), row(g0), row(bt0),
            W1, row(b1), row(g1), row(bt1),
            W2, row(b2), row(g2), row(bt2),
            Wo, row(bo))
    in_specs = ([batch_spec(2 * D), batch_spec(2 * D),
                 batch_spec(1), batch_spec(1)]
                + [full_spec(a) for a in args[4:]])
    return pl.pallas_call(
        _mlp_body,
        grid=grid,
        in_specs=in_specs,
        out_specs=pl.BlockSpec((block_m, 1), lambda i: (i, 0)),
        out_shape=jax.ShapeDtypeStruct((B, 1), jnp.float32),
    )(*args)


def kernel(user_ids, item_ids, user_table, item_table,
           W0, b0, gamma0, beta0,
           W1, b1, gamma1, beta1,
           W2, b2, gamma2, beta2,
           Wo, bo):
    B = user_ids.shape[0]
    info = plsc.get_sparse_core_info()
    n_workers = info.num_cores * info.num_subcores
    chunks = B // (n_workers * IDX_CHUNK)
    upack = _pack_table(user_table.T)
    ipack = _pack_table(item_table.T)
    uid = user_ids.astype(jnp.int32)
    iid = item_ids.astype(jnp.int32)
    n = user_table.shape[0]
    nfull = n // PACK_BLK
    tail_q = (n - nfull * PACK_BLK) // 4
    q = PACK_BLK // 4

    def qrow_sel(i):
        m = i - nfull * PACK_BLK
        row = jnp.where(m < 0, (i // PACK_BLK) * q + i % q,
                        nfull * q + jnp.where(m < 0, 0, m) % max(tail_q, 1))
        sel = jnp.where(m < 0, (i % PACK_BLK) // q,
                        jnp.where(m < 0, 0, m) // max(tail_q, 1))
        return row, sel

    urow, usel = qrow_sel(uid)
    irow, isel = qrow_sel(iid)
    uhi2 = urow.reshape(n_workers * chunks, IDX_CHUNK)
    ihi2 = irow.reshape(n_workers * chunks, IDX_CHUNK)
    uq, iq = _sc_gather_quads(upack, ipack, uhi2, ihi2, n_workers, chunks)
    usel = usel.reshape(B, 1)
    isel = isel.reshape(B, 1)
    out = _mlp(uq, iq, usel, isel,
               W0, b0, gamma0, beta0, W1, b1, gamma1, beta1,
               W2, b2, gamma2, beta2, Wo, bo, block_m=2048)
    return out.reshape(B)
